# baseline (speedup 1.0000x reference)
import functools

import jax
import jax.numpy as jnp
from jax import lax
from jax.experimental import pallas as pl
from jax.experimental.pallas import tpu as pltpu
from jax.experimental.pallas import tpu_sc as plsc

_ANCHORS = [[(10, 13), (16, 30), (33, 23)],
            [(30, 61), (62, 45), (59, 119)],
            [(116, 90), (156, 198), (373, 326)]]
_IMG = 416.0
_NCLS = 5
_B, _A, _C, _N = 64, 3, 10, 32
_HS = (52, 26, 13)
_NW = 32
_IPW = _B // _NW
_NACC = 24
_LN2 = 0.6931471805599453


def _vlog1p(u):
    s = u / (2.0 + u)
    s2 = s * s
    p = 1.0 + s2 * (1.0 / 3.0 + s2 * (0.2 + s2 * (1.0 / 7.0 + s2 * (1.0 / 9.0))))
    return 2.0 * s * p


def _vlog(z):
    bits = plsc.bitcast(z, jnp.int32)
    e = lax.shift_right_arithmetic(bits, 23) - 127
    m = plsc.bitcast((bits & 0x7FFFFF) | 0x3F800000, jnp.float32)
    s = (m - 1.0) / (m + 1.0)
    s2 = s * s
    p = 1.0 + s2 * (1.0 / 3.0 + s2 * (0.2 + s2 * (1.0 / 7.0 + s2 * (1.0 / 9.0))))
    return e.astype(jnp.float32) * _LN2 + 2.0 * s * p


def _vsoftplus(x):
    return jnp.maximum(x, 0.0) + _vlog1p(jnp.exp(-jnp.abs(x)))


def _vsigmoid(x):
    return 1.0 / (1.0 + jnp.exp(-x))


def _sc_sparse(p0f, p1f, p2f, targets):
    mesh = plsc.VectorSubcoreMesh(core_axis_name="c", subcore_axis_name="s",
                                  num_cores=2, num_subcores=16)

    @functools.partial(
        pl.kernel,
        mesh=mesh,
        compiler_params=pltpu.CompilerParams(needs_layout_passes=False),
        out_type=jax.ShapeDtypeStruct((_NW, _NACC, 16), jnp.float32),
        scratch_types=[
            pltpu.VMEM((5, _N, _B), jnp.float32),
            pltpu.VMEM((_N,), jnp.int32),
            pltpu.VMEM((_N,), jnp.int32),
            pltpu.VMEM((_N * 16, _B), jnp.float32),
            pltpu.VMEM((_NACC, 16), jnp.float32),
            pltpu.SemaphoreType.DMA,
        ],
    )
    def body(p0, p1, p2, tgt, out, tgt_v, kbuf, k2buf, rows_v, acc_v, sem):
        wid = lax.axis_index("s") * 2 + lax.axis_index("c")
        iota = lax.iota(jnp.int32, 16)
        zero = jnp.zeros((16,), jnp.float32)
        for a in range(_NACC):
            acc_v[a] = zero
        pltpu.sync_copy(tgt, tgt_v)

        def img_body(i, _carry):
            b = wid * _IPW + i
            bv0 = jnp.full((16,), 0, jnp.int32) + b
            cols = []
            for h in (0, 1):
                rix = iota + 16 * h
                cols.append([plsc.load_gather(
                    tgt_v, [jnp.full((16,), col, jnp.int32), rix, bv0])
                    for col in range(5)])

            for s, (pref, H) in enumerate(((p0, 52), (p1, 26), (p2, 13))):
                W = H
                HW = H * W
                anc = [(aw * W / _IMG, ah * H / _IMG) for (aw, ah) in _ANCHORS[s]]
                base = b * (_A * _C * HW)
                halves = []
                for h in (0, 1):
                    clsf, x, y, w, hh = cols[h]
                    gx = x * float(W)
                    gy = y * float(H)
                    gw = w * float(W)
                    gh = hh * float(H)
                    gi = jnp.clip(gx.astype(jnp.int32), 0, W - 1)
                    gj = jnp.clip(gy.astype(jnp.int32), 0, H - 1)
                    garea = gw * gh
                    best = jnp.zeros((16,), jnp.int32)
                    bi = jnp.full((16,), -1.0, jnp.float32)
                    for k, (aw, ah) in enumerate(anc):
                        ia = jnp.minimum(gw, aw) * jnp.minimum(gh, ah)
                        iou = ia / (garea + aw * ah - ia + 1e-6)
                        if k == 0:
                            best, bi = jnp.zeros((16,), jnp.int32), iou
                        else:
                            best = jnp.where(iou > bi, k, best)
                            bi = jnp.maximum(bi, iou)
                    clsi = clsf.astype(jnp.int32)
                    key = best * HW + gj * W + gi
                    key2 = key * _NCLS + clsi
                    kbuf[pl.ds(16 * h, 16)] = key
                    k2buf[pl.ds(16 * h, 16)] = key2
                    halves.append((gx, gy, gw, gh, gi, gj, best, clsi, key, key2))

                for h in (0, 1):
                    bev = halves[h][6]
                    gjv = halves[h][5]
                    giv = halves[h][4]
                    for t in range(16):
                        pltpu.async_copy(
                            pref.at[bev[t], :, gjv[t], giv[t], :],
                            rows_v.at[pl.ds((16 * h + t) * 16, _C)], sem)

                zi = jnp.zeros((16,), jnp.int32)

                def dd(j, carry):
                    d0, d1, e0, e1 = carry
                    jv = jnp.full((16,), 0, jnp.int32) + j
                    kj = plsc.load_gather(kbuf, [jv])
                    k2j = plsc.load_gather(k2buf, [jv])
                    t0 = iota
                    t1 = iota + 16
                    d0 = d0 | jnp.where((halves[0][8] == kj) & (jv > t0), 1, 0)
                    d1 = d1 | jnp.where((halves[1][8] == kj) & (jv > t1), 1, 0)
                    e0 = e0 | jnp.where((halves[0][9] == k2j) & (jv > t0), 1, 0)
                    e1 = e1 | jnp.where((halves[1][9] == k2j) & (jv > t1), 1, 0)
                    return d0, d1, e0, e1

                d0, d1, e0, e1 = lax.fori_loop(0, _N, dd, (zi, zi, zi, zi))
                dead = (d0, d1)
                dead2 = (e0, e1)

                def drain(t, _c):
                    pltpu.make_async_copy(
                        pref.at[0, :, 0, 0, :],
                        rows_v.at[pl.ds(t * 16, _C)], sem).wait()
                    return 0

                lax.fori_loop(0, _N, drain, 0)

                ab = 8 * s
                for h in (0, 1):
                    gx, gy, gw, gh, gi, gj, best, clsi, key, key2 = halves[h]
                    bv = jnp.full((16,), 0, jnp.int32) + b
                    vals = []
                    for c in range(_C):
                        vals.append(plsc.load_gather(
                            rows_v, [(iota + 16 * h) * 16 + c, bv]))
                    m = jnp.where(dead[h] == 0, 1.0, 0.0)
                    m2 = jnp.where(dead2[h] == 0, 1.0, 0.0)
                    tx = gx - gi.astype(jnp.float32)
                    ty = gy - gj.astype(jnp.float32)
                    bf = best.astype(jnp.float32)
                    aw = jnp.where(best == 0, anc[0][0],
                                   jnp.where(best == 1, anc[1][0], anc[2][0]))
                    ah = jnp.where(best == 0, anc[0][1],
                                   jnp.where(best == 1, anc[1][1], anc[2][1]))
                    del bf
                    twx = _vlog(gw / aw + 1e-6)
                    twy = _vlog(gh / ah + 1e-6)
                    sx = _vsigmoid(vals[0])
                    sy = _vsigmoid(vals[1])
                    dx = sx - tx
                    dy = sy - ty
                    dw = vals[2] - twx
                    dh = vals[3] - twy
                    acc_v[ab + 0] = acc_v[ab + 0] + m * (dx * dx + dy * dy)
                    acc_v[ab + 1] = acc_v[ab + 1] + m * (dw * dw + dh * dh)
                    sp4 = _vsoftplus(vals[4])
                    acc_v[ab + 2] = acc_v[ab + 2] + m * (sp4 - vals[4])
                    acc_v[ab + 3] = acc_v[ab + 3] + m * sp4
                    spc = (_vsoftplus(vals[5]) + _vsoftplus(vals[6]) +
                           _vsoftplus(vals[7]) + _vsoftplus(vals[8]) +
                           _vsoftplus(vals[9]))
                    acc_v[ab + 4] = acc_v[ab + 4] + m * spc
                    pcls = jnp.where(clsi == 0, vals[5],
                                     jnp.where(clsi == 1, vals[6],
                                               jnp.where(clsi == 2, vals[7],
                                                         jnp.where(clsi == 3, vals[8],
                                                                   vals[9]))))
                    acc_v[ab + 5] = acc_v[ab + 5] + m2 * pcls
                    acc_v[ab + 6] = acc_v[ab + 6] + m
            return 0

        lax.fori_loop(0, _IPW, img_body, 0)
        pltpu.sync_copy(acc_v, out.at[wid])

    return body(p0f, p1f, p2f, targets)


def _tc_dense(q0, q1, q2):
    def body(r0, r1, r2, o0, o1, o2):
        @pl.when(pl.program_id(0) == 0)
        def _init():
            o0[0, 0] = 0.0
            o1[0, 0] = 0.0
            o2[0, 0] = 0.0

        for r, o in ((r0, o0), (r1, o1), (r2, o2)):
            x = r[...]
            o[0, 0] += jnp.sum(jnp.maximum(x, 0.0) +
                               jnp.log1p(jnp.exp(-jnp.abs(x))))

    in_specs = [pl.BlockSpec((1, 1, H, H, _B), lambda i: (i, 4, 0, 0, 0))
                for H in _HS]
    out_specs = [pl.BlockSpec(memory_space=pltpu.SMEM)] * 3
    out_shape = [jax.ShapeDtypeStruct((1, 1), jnp.float32)] * 3
    return pl.pallas_call(
        body,
        grid=(_A,),
        in_specs=in_specs,
        out_specs=out_specs,
        out_shape=out_shape,
    )(q0, q1, q2)


def kernel(pred_s0, pred_s1, pred_s2, targets):
    q0 = jnp.transpose(pred_s0, (1, 2, 3, 4, 0))
    q1 = jnp.transpose(pred_s1, (1, 2, 3, 4, 0))
    q2 = jnp.transpose(pred_s2, (1, 2, 3, 4, 0))
    acc = _sc_sparse(q0, q1, q2, jnp.transpose(targets, (2, 1, 0)))
    d0, d1, d2 = _tc_dense(q0, q1, q2)
    dense = jnp.stack([d0[0, 0], d1[0, 0], d2[0, 0]])
    sums = jnp.sum(acc, axis=(0, 2))

    tot = 0.0
    comps = []
    for s, H in enumerate(_HS):
        a = sums[8 * s: 8 * s + 8]
        cnt = a[6]
        cells = float(_B * _A * H * H)
        ncnt = cells - cnt
        lxy = a[0] / (2.0 * cnt)
        lwh = a[1] / (2.0 * cnt)
        lobj = a[2] / cnt
        lcls = (a[4] - a[5]) / (_NCLS * cnt)
        lnoobj = (dense[s] - a[3]) / ncnt * 0.5 * (1.0 - cnt / cells)
        tot = tot + (5.0 * (lxy + lwh) + lobj + lnoobj + lcls) / float(_B * _N)
        comps.append((lxy, lwh, lobj, lnoobj, lcls))

    lxy = comps[0][0] + comps[1][0] + comps[2][0]
    lwh = comps[0][1] + comps[1][1] + comps[2][1]
    lobj = comps[0][2] + comps[1][2] + comps[2][2]
    lnoobj = comps[0][3] + comps[1][3] + comps[2][3]
    lcls = comps[0][4] + comps[1][4] + comps[2][4]
    return jnp.stack([tot / 3.0, lxy, lwh, lobj, lnoobj, lcls])


# speedup vs baseline: 1.0479x; 1.0479x over previous
import functools

import jax
import jax.numpy as jnp
from jax import lax
from jax.experimental import pallas as pl
from jax.experimental.pallas import tpu as pltpu
from jax.experimental.pallas import tpu_sc as plsc

_ANCHORS = [[(10, 13), (16, 30), (33, 23)],
            [(30, 61), (62, 45), (59, 119)],
            [(116, 90), (156, 198), (373, 326)]]
_IMG = 416.0
_NCLS = 5
_B, _A, _C, _N = 64, 3, 10, 32
_HS = (52, 26, 13)
_NW = 32
_IPW = _B // _NW
_NACC = 24
_LN2 = 0.6931471805599453


def _vlog1p(u):
    s = u / (2.0 + u)
    s2 = s * s
    p = 1.0 + s2 * (1.0 / 3.0 + s2 * (0.2 + s2 * (1.0 / 7.0 + s2 * (1.0 / 9.0))))
    return 2.0 * s * p


def _vlog(z):
    bits = plsc.bitcast(z, jnp.int32)
    e = lax.shift_right_arithmetic(bits, 23) - 127
    m = plsc.bitcast((bits & 0x7FFFFF) | 0x3F800000, jnp.float32)
    s = (m - 1.0) / (m + 1.0)
    s2 = s * s
    p = 1.0 + s2 * (1.0 / 3.0 + s2 * (0.2 + s2 * (1.0 / 7.0 + s2 * (1.0 / 9.0))))
    return e.astype(jnp.float32) * _LN2 + 2.0 * s * p


def _vsoftplus(x):
    return jnp.maximum(x, 0.0) + _vlog1p(jnp.exp(-jnp.abs(x)))


def _vsigmoid(x):
    return 1.0 / (1.0 + jnp.exp(-x))


def _sc_sparse(p0f, p1f, p2f, targets):
    mesh = plsc.VectorSubcoreMesh(core_axis_name="c", subcore_axis_name="s",
                                  num_cores=2, num_subcores=16)

    @functools.partial(
        pl.kernel,
        mesh=mesh,
        compiler_params=pltpu.CompilerParams(needs_layout_passes=False),
        out_type=jax.ShapeDtypeStruct((_NW, _NACC, 16), jnp.float32),
        scratch_types=[
            pltpu.VMEM((_N * 5,), jnp.float32),
            pltpu.VMEM((_N,), jnp.int32),
            pltpu.VMEM((_N,), jnp.int32),
            pltpu.VMEM((_N * 16, _B), jnp.float32),
            pltpu.VMEM((_NACC, 16), jnp.float32),
            pltpu.SemaphoreType.DMA,
        ],
    )
    def body(p0, p1, p2, tgt, out, tgt_v, kbuf, k2buf, rows_v, acc_v, sem):
        wid = lax.axis_index("s") * 2 + lax.axis_index("c")
        iota = lax.iota(jnp.int32, 16)
        zero = jnp.zeros((16,), jnp.float32)
        for a in range(_NACC):
            acc_v[a] = zero

        def img_body(i, _carry):
            b = wid * _IPW + i
            pltpu.sync_copy(tgt.at[b], tgt_v)
            cols = []
            for h in (0, 1):
                rix = (iota + 16 * h) * 5
                cols.append([plsc.load_gather(tgt_v, [rix + col])
                             for col in range(5)])

            for s, (pref, H) in enumerate(((p0, 52), (p1, 26), (p2, 13))):
                W = H
                HW = H * W
                anc = [(aw * W / _IMG, ah * H / _IMG) for (aw, ah) in _ANCHORS[s]]
                base = b * (_A * _C * HW)
                halves = []
                for h in (0, 1):
                    clsf, x, y, w, hh = cols[h]
                    gx = x * float(W)
                    gy = y * float(H)
                    gw = w * float(W)
                    gh = hh * float(H)
                    gi = jnp.clip(gx.astype(jnp.int32), 0, W - 1)
                    gj = jnp.clip(gy.astype(jnp.int32), 0, H - 1)
                    garea = gw * gh
                    best = jnp.zeros((16,), jnp.int32)
                    bi = jnp.full((16,), -1.0, jnp.float32)
                    for k, (aw, ah) in enumerate(anc):
                        ia = jnp.minimum(gw, aw) * jnp.minimum(gh, ah)
                        iou = ia / (garea + aw * ah - ia + 1e-6)
                        if k == 0:
                            best, bi = jnp.zeros((16,), jnp.int32), iou
                        else:
                            best = jnp.where(iou > bi, k, best)
                            bi = jnp.maximum(bi, iou)
                    clsi = clsf.astype(jnp.int32)
                    key = best * HW + gj * W + gi
                    key2 = key * _NCLS + clsi
                    kbuf[pl.ds(16 * h, 16)] = key
                    k2buf[pl.ds(16 * h, 16)] = key2
                    halves.append((gx, gy, gw, gh, gi, gj, best, clsi, key, key2))

                for h in (0, 1):
                    bev = halves[h][6]
                    gjv = halves[h][5]
                    giv = halves[h][4]
                    for t in range(16):
                        pltpu.async_copy(
                            pref.at[bev[t], :, gjv[t], giv[t], :],
                            rows_v.at[pl.ds((16 * h + t) * 16, _C)], sem)

                zi = jnp.zeros((16,), jnp.int32)

                def dd(j, carry):
                    d0, d1, e0, e1 = carry
                    jv = jnp.full((16,), 0, jnp.int32) + j
                    kj = plsc.load_gather(kbuf, [jv])
                    k2j = plsc.load_gather(k2buf, [jv])
                    t0 = iota
                    t1 = iota + 16
                    d0 = d0 | jnp.where((halves[0][8] == kj) & (jv > t0), 1, 0)
                    d1 = d1 | jnp.where((halves[1][8] == kj) & (jv > t1), 1, 0)
                    e0 = e0 | jnp.where((halves[0][9] == k2j) & (jv > t0), 1, 0)
                    e1 = e1 | jnp.where((halves[1][9] == k2j) & (jv > t1), 1, 0)
                    return d0, d1, e0, e1

                d0, d1, e0, e1 = lax.fori_loop(0, _N, dd, (zi, zi, zi, zi))
                dead = (d0, d1)
                dead2 = (e0, e1)

                def drain(t, _c):
                    pltpu.make_async_copy(
                        pref.at[0, :, 0, 0, :],
                        rows_v.at[pl.ds(t * 16, _C)], sem).wait()
                    return 0

                lax.fori_loop(0, _N, drain, 0)

                ab = 8 * s
                for h in (0, 1):
                    gx, gy, gw, gh, gi, gj, best, clsi, key, key2 = halves[h]
                    bv = jnp.full((16,), 0, jnp.int32) + b
                    vals = []
                    for c in range(_C):
                        vals.append(plsc.load_gather(
                            rows_v, [(iota + 16 * h) * 16 + c, bv]))
                    m = jnp.where(dead[h] == 0, 1.0, 0.0)
                    m2 = jnp.where(dead2[h] == 0, 1.0, 0.0)
                    tx = gx - gi.astype(jnp.float32)
                    ty = gy - gj.astype(jnp.float32)
                    bf = best.astype(jnp.float32)
                    aw = jnp.where(best == 0, anc[0][0],
                                   jnp.where(best == 1, anc[1][0], anc[2][0]))
                    ah = jnp.where(best == 0, anc[0][1],
                                   jnp.where(best == 1, anc[1][1], anc[2][1]))
                    del bf
                    twx = _vlog(gw / aw + 1e-6)
                    twy = _vlog(gh / ah + 1e-6)
                    sx = _vsigmoid(vals[0])
                    sy = _vsigmoid(vals[1])
                    dx = sx - tx
                    dy = sy - ty
                    dw = vals[2] - twx
                    dh = vals[3] - twy
                    acc_v[ab + 0] = acc_v[ab + 0] + m * (dx * dx + dy * dy)
                    acc_v[ab + 1] = acc_v[ab + 1] + m * (dw * dw + dh * dh)
                    sp4 = _vsoftplus(vals[4])
                    acc_v[ab + 2] = acc_v[ab + 2] + m * (sp4 - vals[4])
                    acc_v[ab + 3] = acc_v[ab + 3] + m * sp4
                    spc = (_vsoftplus(vals[5]) + _vsoftplus(vals[6]) +
                           _vsoftplus(vals[7]) + _vsoftplus(vals[8]) +
                           _vsoftplus(vals[9]))
                    acc_v[ab + 4] = acc_v[ab + 4] + m * spc
                    pcls = jnp.where(clsi == 0, vals[5],
                                     jnp.where(clsi == 1, vals[6],
                                               jnp.where(clsi == 2, vals[7],
                                                         jnp.where(clsi == 3, vals[8],
                                                                   vals[9]))))
                    acc_v[ab + 5] = acc_v[ab + 5] + m2 * pcls
                    acc_v[ab + 6] = acc_v[ab + 6] + m
            return 0

        lax.fori_loop(0, _IPW, img_body, 0)
        pltpu.sync_copy(acc_v, out.at[wid])

    return body(p0f, p1f, p2f, targets)


def _tc_dense(q0, q1, q2):
    def body(r0, r1, r2, o0, o1, o2):
        @pl.when(pl.program_id(0) == 0)
        def _init():
            o0[0, 0] = 0.0
            o1[0, 0] = 0.0
            o2[0, 0] = 0.0

        for r, o in ((r0, o0), (r1, o1), (r2, o2)):
            x = r[...]
            o[0, 0] += jnp.sum(jnp.maximum(x, 0.0) +
                               jnp.log1p(jnp.exp(-jnp.abs(x))))

    in_specs = [pl.BlockSpec((1, 1, H, H, _B), lambda i: (i, 4, 0, 0, 0))
                for H in _HS]
    out_specs = [pl.BlockSpec(memory_space=pltpu.SMEM)] * 3
    out_shape = [jax.ShapeDtypeStruct((1, 1), jnp.float32)] * 3
    return pl.pallas_call(
        body,
        grid=(_A,),
        in_specs=in_specs,
        out_specs=out_specs,
        out_shape=out_shape,
    )(q0, q1, q2)


def kernel(pred_s0, pred_s1, pred_s2, targets):
    q0 = jnp.transpose(pred_s0, (1, 2, 3, 4, 0))
    q1 = jnp.transpose(pred_s1, (1, 2, 3, 4, 0))
    q2 = jnp.transpose(pred_s2, (1, 2, 3, 4, 0))
    acc = _sc_sparse(q0, q1, q2, targets.reshape(_B, _N * 5))
    d0, d1, d2 = _tc_dense(q0, q1, q2)
    dense = jnp.stack([d0[0, 0], d1[0, 0], d2[0, 0]])
    sums = jnp.sum(acc, axis=(0, 2))

    tot = 0.0
    comps = []
    for s, H in enumerate(_HS):
        a = sums[8 * s: 8 * s + 8]
        cnt = a[6]
        cells = float(_B * _A * H * H)
        ncnt = cells - cnt
        lxy = a[0] / (2.0 * cnt)
        lwh = a[1] / (2.0 * cnt)
        lobj = a[2] / cnt
        lcls = (a[4] - a[5]) / (_NCLS * cnt)
        lnoobj = (dense[s] - a[3]) / ncnt * 0.5 * (1.0 - cnt / cells)
        tot = tot + (5.0 * (lxy + lwh) + lobj + lnoobj + lcls) / float(_B * _N)
        comps.append((lxy, lwh, lobj, lnoobj, lcls))

    lxy = comps[0][0] + comps[1][0] + comps[2][0]
    lwh = comps[0][1] + comps[1][1] + comps[2][1]
    lobj = comps[0][2] + comps[1][2] + comps[2][2]
    lnoobj = comps[0][3] + comps[1][3] + comps[2][3]
    lcls = comps[0][4] + comps[1][4] + comps[2][4]
    return jnp.stack([tot / 3.0, lxy, lwh, lobj, lnoobj, lcls])


# speedup vs baseline: 1.1220x; 1.0707x over previous
import functools

import jax
import jax.numpy as jnp
from jax import lax
from jax.experimental import pallas as pl
from jax.experimental.pallas import tpu as pltpu
from jax.experimental.pallas import tpu_sc as plsc

_ANCHORS = [[(10, 13), (16, 30), (33, 23)],
            [(30, 61), (62, 45), (59, 119)],
            [(116, 90), (156, 198), (373, 326)]]
_IMG = 416.0
_NCLS = 5
_B, _A, _C, _N = 64, 3, 10, 32
_HS = (52, 26, 13)
_NW = 32
_IPW = _B // _NW
_NACC = 24
_LN2 = 0.6931471805599453


def _vlog1p(u):
    s = u / (2.0 + u)
    s2 = s * s
    p = 1.0 + s2 * (1.0 / 3.0 + s2 * (0.2 + s2 * (1.0 / 7.0 + s2 * (1.0 / 9.0))))
    return 2.0 * s * p


def _vlog(z):
    bits = plsc.bitcast(z, jnp.int32)
    e = lax.shift_right_arithmetic(bits, 23) - 127
    m = plsc.bitcast((bits & 0x7FFFFF) | 0x3F800000, jnp.float32)
    s = (m - 1.0) / (m + 1.0)
    s2 = s * s
    p = 1.0 + s2 * (1.0 / 3.0 + s2 * (0.2 + s2 * (1.0 / 7.0 + s2 * (1.0 / 9.0))))
    return e.astype(jnp.float32) * _LN2 + 2.0 * s * p


def _vsoftplus(x):
    return jnp.maximum(x, 0.0) + _vlog1p(jnp.exp(-jnp.abs(x)))


def _vsigmoid(x):
    return 1.0 / (1.0 + jnp.exp(-x))


def _sc_sparse(p0f, p1f, p2f, targets):
    mesh = plsc.VectorSubcoreMesh(core_axis_name="c", subcore_axis_name="s",
                                  num_cores=2, num_subcores=16)

    @functools.partial(
        pl.kernel,
        mesh=mesh,
        compiler_params=pltpu.CompilerParams(needs_layout_passes=False),
        out_type=jax.ShapeDtypeStruct((_NW, _NACC, 16), jnp.float32),
        scratch_types=[
            pltpu.VMEM((_N * 5,), jnp.float32),
            pltpu.VMEM((3, _N), jnp.int32),
            pltpu.VMEM((3, _N), jnp.int32),
            [pltpu.VMEM((_N * _C, _B), jnp.float32) for _ in range(2)],
            pltpu.VMEM((_NACC, 16), jnp.float32),
            pltpu.SemaphoreType.DMA,
        ],
    )
    def body(p0, p1, p2, tgt, out, tgt_v, kbuf, k2buf, rows_v, acc_v, sem):
        wid = lax.axis_index("s") * 2 + lax.axis_index("c")
        iota = lax.iota(jnp.int32, 16)
        zero = jnp.zeros((16,), jnp.float32)
        for a in range(_NACC):
            acc_v[a] = zero

        def img_body(i, _carry):
            b = wid * _IPW + i
            pltpu.sync_copy(tgt.at[b], tgt_v)
            cols = []
            for h in (0, 1):
                rix = (iota + 16 * h) * 5
                cols.append([plsc.load_gather(tgt_v, [rix + col])
                             for col in range(5)])

            scales = ((p0, 52), (p1, 26), (p2, 13))

            def phase1(s, buf):
                pref, H = scales[s]
                W = H
                HW = H * W
                anc = [(aw * W / _IMG, ah * H / _IMG) for (aw, ah) in _ANCHORS[s]]
                halves = []
                for h in (0, 1):
                    clsf, x, y, w, hh = cols[h]
                    gx = x * float(W)
                    gy = y * float(H)
                    gw = w * float(W)
                    gh = hh * float(H)
                    gi = jnp.clip(gx.astype(jnp.int32), 0, W - 1)
                    gj = jnp.clip(gy.astype(jnp.int32), 0, H - 1)
                    garea = gw * gh
                    best = jnp.zeros((16,), jnp.int32)
                    bi = jnp.full((16,), -1.0, jnp.float32)
                    for k, (aw, ah) in enumerate(anc):
                        ia = jnp.minimum(gw, aw) * jnp.minimum(gh, ah)
                        iou = ia / (garea + aw * ah - ia + 1e-6)
                        if k == 0:
                            best, bi = jnp.zeros((16,), jnp.int32), iou
                        else:
                            best = jnp.where(iou > bi, k, best)
                            bi = jnp.maximum(bi, iou)
                    clsi = clsf.astype(jnp.int32)
                    key = best * HW + gj * W + gi
                    key2 = key * _NCLS + clsi
                    kbuf[s, pl.ds(16 * h, 16)] = key
                    k2buf[s, pl.ds(16 * h, 16)] = key2
                    halves.append((gx, gy, gw, gh, gi, gj, best, clsi, key, key2))

                for h in (0, 1):
                    bev = halves[h][6]
                    gjv = halves[h][5]
                    giv = halves[h][4]
                    for t in range(16):
                        pltpu.async_copy(
                            pref.at[bev[t], :, gjv[t], giv[t], :],
                            buf.at[pl.ds((16 * h + t) * _C, _C)], sem)
                return halves

            def phase2(s, halves, buf):
                pref, H = scales[s]
                W = H
                anc = [(aw * W / _IMG, ah * H / _IMG) for (aw, ah) in _ANCHORS[s]]
                zi = jnp.zeros((16,), jnp.int32)
                sv = jnp.full((16,), 0, jnp.int32) + s

                def dd(j, carry):
                    d0, d1, e0, e1 = carry
                    jv = jnp.full((16,), 0, jnp.int32) + j
                    kj = plsc.load_gather(kbuf, [sv, jv])
                    k2j = plsc.load_gather(k2buf, [sv, jv])
                    t0 = iota
                    t1 = iota + 16
                    d0 = d0 | jnp.where((halves[0][8] == kj) & (jv > t0), 1, 0)
                    d1 = d1 | jnp.where((halves[1][8] == kj) & (jv > t1), 1, 0)
                    e0 = e0 | jnp.where((halves[0][9] == k2j) & (jv > t0), 1, 0)
                    e1 = e1 | jnp.where((halves[1][9] == k2j) & (jv > t1), 1, 0)
                    return d0, d1, e0, e1

                d0, d1, e0, e1 = lax.fori_loop(0, _N, dd, (zi, zi, zi, zi))
                dead = (d0, d1)
                dead2 = (e0, e1)

                def drain(t, _c):
                    pltpu.make_async_copy(
                        pref.at[0, :, 0, 0, :],
                        buf.at[pl.ds(t * _C, _C)], sem).wait()
                    return 0

                lax.fori_loop(0, _N, drain, 0)

                ab = 8 * s
                for h in (0, 1):
                    gx, gy, gw, gh, gi, gj, best, clsi, key, key2 = halves[h]
                    bv = jnp.full((16,), 0, jnp.int32) + b
                    vals = []
                    for c in range(_C):
                        vals.append(plsc.load_gather(
                            buf, [(iota + 16 * h) * _C + c, bv]))
                    m = jnp.where(dead[h] == 0, 1.0, 0.0)
                    m2 = jnp.where(dead2[h] == 0, 1.0, 0.0)
                    tx = gx - gi.astype(jnp.float32)
                    ty = gy - gj.astype(jnp.float32)
                    bf = best.astype(jnp.float32)
                    aw = jnp.where(best == 0, anc[0][0],
                                   jnp.where(best == 1, anc[1][0], anc[2][0]))
                    ah = jnp.where(best == 0, anc[0][1],
                                   jnp.where(best == 1, anc[1][1], anc[2][1]))
                    del bf
                    twx = _vlog(gw / aw + 1e-6)
                    twy = _vlog(gh / ah + 1e-6)
                    sx = _vsigmoid(vals[0])
                    sy = _vsigmoid(vals[1])
                    dx = sx - tx
                    dy = sy - ty
                    dw = vals[2] - twx
                    dh = vals[3] - twy
                    acc_v[ab + 0] = acc_v[ab + 0] + m * (dx * dx + dy * dy)
                    acc_v[ab + 1] = acc_v[ab + 1] + m * (dw * dw + dh * dh)
                    sp4 = _vsoftplus(vals[4])
                    acc_v[ab + 2] = acc_v[ab + 2] + m * (sp4 - vals[4])
                    acc_v[ab + 3] = acc_v[ab + 3] + m * sp4
                    spc = (_vsoftplus(vals[5]) + _vsoftplus(vals[6]) +
                           _vsoftplus(vals[7]) + _vsoftplus(vals[8]) +
                           _vsoftplus(vals[9]))
                    acc_v[ab + 4] = acc_v[ab + 4] + m * spc
                    pcls = jnp.where(clsi == 0, vals[5],
                                     jnp.where(clsi == 1, vals[6],
                                               jnp.where(clsi == 2, vals[7],
                                                         jnp.where(clsi == 3, vals[8],
                                                                   vals[9]))))
                    acc_v[ab + 5] = acc_v[ab + 5] + m2 * pcls
                    acc_v[ab + 6] = acc_v[ab + 6] + m

            h0 = phase1(0, rows_v[0])
            h1 = phase1(1, rows_v[1])
            phase2(0, h0, rows_v[0])
            h2 = phase1(2, rows_v[0])
            phase2(1, h1, rows_v[1])
            phase2(2, h2, rows_v[0])
            return 0

        lax.fori_loop(0, _IPW, img_body, 0)
        pltpu.sync_copy(acc_v, out.at[wid])

    return body(p0f, p1f, p2f, targets)


def _tc_dense(q0, q1, q2):
    def body(r0, r1, r2, o0, o1, o2):
        @pl.when(pl.program_id(0) == 0)
        def _init():
            o0[0, 0] = 0.0
            o1[0, 0] = 0.0
            o2[0, 0] = 0.0

        for r, o in ((r0, o0), (r1, o1), (r2, o2)):
            x = r[...]
            o[0, 0] += jnp.sum(jnp.maximum(x, 0.0) +
                               jnp.log1p(jnp.exp(-jnp.abs(x))))

    in_specs = [pl.BlockSpec((1, 1, H, H, _B), lambda i: (i, 4, 0, 0, 0))
                for H in _HS]
    out_specs = [pl.BlockSpec(memory_space=pltpu.SMEM)] * 3
    out_shape = [jax.ShapeDtypeStruct((1, 1), jnp.float32)] * 3
    return pl.pallas_call(
        body,
        grid=(_A,),
        in_specs=in_specs,
        out_specs=out_specs,
        out_shape=out_shape,
    )(q0, q1, q2)


def kernel(pred_s0, pred_s1, pred_s2, targets):
    q0 = jnp.transpose(pred_s0, (1, 2, 3, 4, 0))
    q1 = jnp.transpose(pred_s1, (1, 2, 3, 4, 0))
    q2 = jnp.transpose(pred_s2, (1, 2, 3, 4, 0))
    acc = _sc_sparse(q0, q1, q2, targets.reshape(_B, _N * 5))
    d0, d1, d2 = _tc_dense(q0, q1, q2)
    dense = jnp.stack([d0[0, 0], d1[0, 0], d2[0, 0]])
    sums = jnp.sum(acc, axis=(0, 2))

    tot = 0.0
    comps = []
    for s, H in enumerate(_HS):
        a = sums[8 * s: 8 * s + 8]
        cnt = a[6]
        cells = float(_B * _A * H * H)
        ncnt = cells - cnt
        lxy = a[0] / (2.0 * cnt)
        lwh = a[1] / (2.0 * cnt)
        lobj = a[2] / cnt
        lcls = (a[4] - a[5]) / (_NCLS * cnt)
        lnoobj = (dense[s] - a[3]) / ncnt * 0.5 * (1.0 - cnt / cells)
        tot = tot + (5.0 * (lxy + lwh) + lobj + lnoobj + lcls) / float(_B * _N)
        comps.append((lxy, lwh, lobj, lnoobj, lcls))

    lxy = comps[0][0] + comps[1][0] + comps[2][0]
    lwh = comps[0][1] + comps[1][1] + comps[2][1]
    lobj = comps[0][2] + comps[1][2] + comps[2][2]
    lnoobj = comps[0][3] + comps[1][3] + comps[2][3]
    lcls = comps[0][4] + comps[1][4] + comps[2][4]
    return jnp.stack([tot / 3.0, lxy, lwh, lobj, lnoobj, lcls])
